# baseline (speedup 1.0000x reference)
import functools

import jax
import jax.numpy as jnp
import numpy as np
from jax.experimental import pallas as pl
from jax.experimental.pallas import tpu as pltpu

_THR = 0.75
_NOC = -1


def _nn_kernel(x_ref, c_ref, a_ref, o_ref, c2m_ref, min_ref, arg_ref,
               *, nt, bn):
    i = pl.program_id(0)
    j = pl.program_id(1)

    @pl.when(j == 0)
    def _row_init():
        min_ref[...] = jnp.full_like(min_ref, jnp.inf)
        arg_ref[...] = jnp.zeros_like(arg_ref)

    @pl.when(i == 0)
    def _code_init():
        c = c_ref[...]
        c2 = jnp.sum(c * c, axis=1)[None, :]
        c2m_ref[j] = jnp.where(a_ref[...] > 0, c2, jnp.inf)

    s = jax.lax.dot_general(x_ref[...], c_ref[...], (((1,), (1,)), ((), ())),
                            preferred_element_type=jnp.float32)
    v = s + c2m_ref[j]
    tmin = jnp.min(v, axis=1, keepdims=True)
    iota = jax.lax.broadcasted_iota(jnp.int32, v.shape, 1)
    targ = jnp.min(jnp.where(v == tmin, iota, bn), axis=1, keepdims=True) + j * bn
    better = tmin < min_ref[...]
    arg_ref[...] = jnp.where(better, targ, arg_ref[...])
    min_ref[...] = jnp.where(better, tmin, min_ref[...])

    @pl.when(j == nt - 1)
    def _fin():
        xs = x_ref[...]
        x2 = jnp.sum(xs * xs, axis=1, keepdims=True) * 0.25
        mind = min_ref[...] + x2
        o_ref[...] = jnp.where(mind <= _THR, arg_ref[...], _NOC).astype(jnp.int32)


def kernel(imgs, patch_size, codes, active):
    B, C, T, H, W = imgs.shape
    N, D = codes.shape
    p = int(np.sqrt(D // C))
    Hp, Wp = H // p, W // p
    x = imgs.reshape(B, C, T, Hp, p, Wp, p).transpose(0, 2, 3, 5, 4, 6, 1)
    xs = (x * -2.0).reshape(-1, D)
    M = xs.shape[0]

    BN = 512
    BM = next((b for b in (1536, 1152, 768, 512, 256, 128, 8) if M % b == 0), M)
    MT, NT = M // BM, N // BN
    amask = active.astype(jnp.float32).reshape(1, N)

    out = pl.pallas_call(
        functools.partial(_nn_kernel, nt=NT, bn=BN),
        grid=(MT, NT),
        in_specs=[
            pl.BlockSpec((BM, D), lambda i, j: (i, 0)),
            pl.BlockSpec((BN, D), lambda i, j: (j, 0)),
            pl.BlockSpec((1, BN), lambda i, j: (0, j)),
        ],
        out_specs=pl.BlockSpec((BM, 1), lambda i, j: (i, 0)),
        out_shape=jax.ShapeDtypeStruct((M, 1), jnp.int32),
        scratch_shapes=[
            pltpu.VMEM((NT, 1, BN), jnp.float32),
            pltpu.VMEM((BM, 1), jnp.float32),
            pltpu.VMEM((BM, 1), jnp.int32),
        ],
    )(xs, codes, amask)
    return out.reshape(B, T, Hp, Wp)


# speedup vs baseline: 1.0463x; 1.0463x over previous
import functools

import jax
import jax.numpy as jnp
import numpy as np
from jax.experimental import pallas as pl
from jax.experimental.pallas import tpu as pltpu

_THR = 0.75
_NOC = -1


def _nn_kernel(x_ref, c_ref, a_ref, o_ref, c2m_ref, min_ref, arg_ref,
               *, nt, bn):
    i = pl.program_id(0)
    j = pl.program_id(1)

    @pl.when(j == 0)
    def _row_init():
        min_ref[...] = jnp.full_like(min_ref, jnp.inf)
        arg_ref[...] = jnp.zeros_like(arg_ref)

    @pl.when(i == 0)
    def _code_init():
        c = c_ref[...]
        c2 = jnp.sum(c * c, axis=1)[None, :]
        c2m_ref[j] = jnp.where(a_ref[...] > 0, c2, jnp.inf)

    s = jax.lax.dot_general(x_ref[...], c_ref[...], (((1,), (1,)), ((), ())),
                            preferred_element_type=jnp.float32)
    v = s + c2m_ref[j]
    tmin = jnp.min(v, axis=1, keepdims=True)
    iota = jax.lax.broadcasted_iota(jnp.int32, v.shape, 1)
    targ = jnp.min(jnp.where(v == tmin, iota, bn), axis=1, keepdims=True) + j * bn
    better = tmin < min_ref[...]
    arg_ref[...] = jnp.where(better, targ, arg_ref[...])
    min_ref[...] = jnp.where(better, tmin, min_ref[...])

    @pl.when(j == nt - 1)
    def _fin():
        xs = x_ref[...]
        x2 = jnp.sum(xs * xs, axis=1, keepdims=True) * 0.25
        mind = min_ref[...] + x2
        o_ref[...] = jnp.where(mind <= _THR, arg_ref[...], _NOC).astype(jnp.int32)


def kernel(imgs, patch_size, codes, active):
    B, C, T, H, W = imgs.shape
    N, D = codes.shape
    p = int(np.sqrt(D // C))
    Hp, Wp = H // p, W // p
    x = imgs.reshape(B, C, T, Hp, p, Wp, p).transpose(0, 2, 3, 5, 4, 6, 1)
    xs = (x * -2.0).reshape(-1, D)
    M = xs.shape[0]

    BN = 512
    BM = next((b for b in (4608, 1536, 1152, 768, 512, 256, 128, 8) if M % b == 0), M)
    MT, NT = M // BM, N // BN
    amask = active.astype(jnp.float32).reshape(1, N)

    out = pl.pallas_call(
        functools.partial(_nn_kernel, nt=NT, bn=BN),
        grid=(MT, NT),
        in_specs=[
            pl.BlockSpec((BM, D), lambda i, j: (i, 0)),
            pl.BlockSpec((BN, D), lambda i, j: (j, 0)),
            pl.BlockSpec((1, BN), lambda i, j: (0, j)),
        ],
        out_specs=pl.BlockSpec((BM, 1), lambda i, j: (i, 0)),
        out_shape=jax.ShapeDtypeStruct((M, 1), jnp.int32),
        scratch_shapes=[
            pltpu.VMEM((NT, 1, BN), jnp.float32),
            pltpu.VMEM((BM, 1), jnp.float32),
            pltpu.VMEM((BM, 1), jnp.int32),
        ],
    )(xs, codes, amask)
    return out.reshape(B, T, Hp, Wp)


# speedup vs baseline: 1.8511x; 1.7692x over previous
import functools

import jax
import jax.numpy as jnp
import numpy as np
from jax.experimental import pallas as pl
from jax.experimental.pallas import tpu as pltpu

_THR = 0.75
_NOC = -1


def _patchify_kernel(img_ref, p_ref, o_ref, *, C, Hp, Wp, p):
    img = img_ref[0, :, 0]
    pieces = []
    for c in range(C):
        v = img[c].reshape(Hp, p, Wp, p)
        v = v.transpose(0, 2, 1, 3)
        pieces.append(v.reshape(Hp * Wp, p * p))
    chunk = jnp.concatenate(pieces, axis=1)
    o_ref[...] = jax.lax.dot_general(
        chunk, p_ref[...], (((1,), (0,)), ((), ())),
        preferred_element_type=jnp.float32)


def _nn_kernel(x_ref, c_ref, a_ref, o_ref, min_ref, arg_ref, *, nt, bn):
    j = pl.program_id(0)

    @pl.when(j == 0)
    def _init():
        min_ref[...] = jnp.full_like(min_ref, jnp.inf)
        arg_ref[...] = jnp.zeros_like(arg_ref)

    c = c_ref[...]
    c2 = jnp.sum(c * c, axis=1)[None, :]
    c2m = jnp.where(a_ref[...] > 0, c2, jnp.inf)

    s = jax.lax.dot_general(x_ref[...], c, (((1,), (1,)), ((), ())),
                            preferred_element_type=jnp.float32)
    v = s + c2m
    tmin = jnp.min(v, axis=1, keepdims=True)
    iota = jax.lax.broadcasted_iota(jnp.int32, v.shape, 1)
    targ = jnp.min(jnp.where(v == tmin, iota, bn), axis=1, keepdims=True) + j * bn
    better = tmin < min_ref[...]
    arg_ref[...] = jnp.where(better, targ, arg_ref[...])
    min_ref[...] = jnp.where(better, tmin, min_ref[...])

    @pl.when(j == nt - 1)
    def _fin():
        xs = x_ref[...]
        x2 = jnp.sum(xs * xs, axis=1, keepdims=True) * 0.25
        mind = min_ref[...] + x2
        o_ref[...] = jnp.where(mind <= _THR, arg_ref[...], _NOC).astype(jnp.int32)


def _perm_matrix(p, C):
    D = p * p * C
    P = np.zeros((D, D), np.float32)
    for c in range(C):
        for q in range(p * p):
            P[c * p * p + q, q * C + c] = -2.0
    return jnp.asarray(P)


def kernel(imgs, patch_size, codes, active):
    B, C, T, H, W = imgs.shape
    N, D = codes.shape
    p = int(np.sqrt(D // C))
    Hp, Wp = H // p, W // p
    M = B * T * Hp * Wp
    R = Hp * Wp

    img6 = imgs.reshape(B, C, T, Hp, p, Wp * p)
    P = _perm_matrix(p, C)

    xs = pl.pallas_call(
        functools.partial(_patchify_kernel, C=C, Hp=Hp, Wp=Wp, p=p),
        grid=(B * T,),
        in_specs=[
            pl.BlockSpec((1, C, 1, Hp, p, Wp * p),
                         lambda k: (k // T, 0, k % T, 0, 0, 0)),
            pl.BlockSpec((D, D), lambda k: (0, 0)),
        ],
        out_specs=pl.BlockSpec((R, D), lambda k: (k, 0)),
        out_shape=jax.ShapeDtypeStruct((M, D), jnp.float32),
    )(img6, P)

    BN = 512
    NT = N // BN
    amask = active.astype(jnp.float32).reshape(1, N)

    out = pl.pallas_call(
        functools.partial(_nn_kernel, nt=NT, bn=BN),
        grid=(NT,),
        in_specs=[
            pl.BlockSpec((M, D), lambda j: (0, 0)),
            pl.BlockSpec((BN, D), lambda j: (j, 0)),
            pl.BlockSpec((1, BN), lambda j: (0, j)),
        ],
        out_specs=pl.BlockSpec((M, 1), lambda j: (0, 0)),
        out_shape=jax.ShapeDtypeStruct((M, 1), jnp.int32),
        scratch_shapes=[
            pltpu.VMEM((M, 1), jnp.float32),
            pltpu.VMEM((M, 1), jnp.int32),
        ],
    )(xs, codes, amask)
    return out.reshape(B, T, Hp, Wp)
